# Initial kernel scaffold; baseline (speedup 1.0000x reference)
import jax
import jax.numpy as jnp
from jax.experimental import pallas as pl


def kernel(x, a, e, Ws1, bs1, Wai1, bai1, Wao1, bao1, Wn1, bn1, We1, be1, Ws2, bs2, Wai2, bai2, Wao2, bao2, Wn2, bn2, We2, be2, Wd, bd):
    raise NotImplementedError("write your pallas kernel here")


# speedup vs baseline: 4.2683x; 4.2683x over previous
import jax
import jax.numpy as jnp
from jax.experimental import pallas as pl

N = 512
K = 32
TI = 64


def _net_kernel(x_ref, a_ref, e_ref, eT_ref,
                Ws1_ref, bs1_ref, Wai1_ref, bai1_ref, Wao1_ref, bao1_ref,
                Wn1_ref, bn1_ref, We1_ref, be1_ref,
                Ws2_ref, bs2_ref, Wai2_ref, bai2_ref, Wao2_ref, bao2_ref,
                Wn2_ref, bn2_ref, Wd_ref, bd_ref,
                y_ref):
    x = x_ref[...]
    mask = (a_ref[...] != 0.0).astype(jnp.float32)

    def xenet(x_arr, e_arr, eT_arr, Ws, bs, Wai, bai, Wao, bao,
              We, be, want_e_new):
        f_in = x_arr.shape[1]
        u = jnp.dot(x_arr, Ws[:f_in, :],
                    preferred_element_type=jnp.float32) + bs
        vT = jax.lax.dot_general(Ws[f_in:2 * f_in, :], x_arr,
                                 (((0,), (1,)), ((), ())),
                                 preferred_element_type=jnp.float32)
        p = Ws[2 * f_in:2 * f_in + 1, :].reshape(1, K, 1)
        q = Ws[2 * f_in + 1:2 * f_in + 2, :].reshape(1, K, 1)
        wai = Wai.reshape(1, K, 1)
        wao = Wao.reshape(1, K, 1)
        we = We.reshape(1, K, 1) if want_e_new else None

        m_in_parts = []
        e_new_parts = []
        m_out = jnp.zeros((K, N), jnp.float32)
        for s in range(N // TI):
            lo, hi = s * TI, (s + 1) * TI
            z = (u[lo:hi][:, :, None] + vT[None, :, :]
                 + p * e_arr[lo:hi][:, None, :]
                 + q * eT_arr[lo:hi][:, None, :])
            t = jnp.maximum(z, 0.0)
            ai = jax.nn.sigmoid(jnp.sum(t * wai, axis=1) + bai)
            ao = jax.nn.sigmoid(jnp.sum(t * wao, axis=1) + bao)
            mk = mask[lo:hi]
            m_in_parts.append(jnp.sum(t * (mk * ai)[:, None, :], axis=2))
            m_out = m_out + jnp.sum(t * (mk * ao)[:, None, :], axis=0)
            if want_e_new:
                e_new_parts.append(jnp.sum(t * we, axis=1) + be)
        m_in = jnp.concatenate(m_in_parts, axis=0)
        e_new = (jnp.concatenate(e_new_parts, axis=0)
                 if want_e_new else None)
        return m_in, m_out, e_new

    def node_update(x_arr, m_in, m_out, Wn, bn):
        f_in = x_arr.shape[1]
        out = jnp.dot(x_arr, Wn[:f_in, :],
                      preferred_element_type=jnp.float32)
        out = out + jnp.dot(m_in, Wn[f_in:f_in + K, :],
                            preferred_element_type=jnp.float32)
        out = out + jax.lax.dot_general(m_out, Wn[f_in + K:f_in + 2 * K, :],
                                        (((0,), (0,)), ((), ())),
                                        preferred_element_type=jnp.float32)
        return out + bn

    m_in1, m_out1, e1 = xenet(x, e_ref[...], eT_ref[...],
                              Ws1_ref[...], bs1_ref[...],
                              Wai1_ref[...], bai1_ref[...],
                              Wao1_ref[...], bao1_ref[...],
                              We1_ref[...], be1_ref[...], True)
    x1 = node_update(x, m_in1, m_out1, Wn1_ref[...], bn1_ref[...])
    e1T = e1.T

    m_in2, m_out2, _ = xenet(x1, e1, e1T,
                             Ws2_ref[...], bs2_ref[...],
                             Wai2_ref[...], bai2_ref[...],
                             Wao2_ref[...], bao2_ref[...],
                             None, None, False)
    x2 = node_update(x1, m_in2, m_out2, Wn2_ref[...], bn2_ref[...])

    y_ref[...] = jnp.dot(x2, Wd_ref[...],
                         preferred_element_type=jnp.float32) + bd_ref[...]


def kernel(x, a, e, Ws1, bs1, Wai1, bai1, Wao1, bao1, Wn1, bn1, We1, be1,
           Ws2, bs2, Wai2, bai2, Wao2, bao2, Wn2, bn2, We2, be2, Wd, bd):
    del We2, be2
    x0 = x[0]
    a0 = a[0]
    e0 = e[0, :, :, 0]
    eT0 = jnp.swapaxes(e0, 0, 1)
    args = (x0, a0, e0, eT0,
            Ws1, bs1.reshape(1, -1), Wai1, bai1.reshape(1, 1),
            Wao1, bao1.reshape(1, 1), Wn1, bn1.reshape(1, -1),
            We1, be1.reshape(1, 1),
            Ws2, bs2.reshape(1, -1), Wai2, bai2.reshape(1, 1),
            Wao2, bao2.reshape(1, 1), Wn2, bn2.reshape(1, -1),
            Wd, bd.reshape(1, -1))
    out = pl.pallas_call(
        _net_kernel,
        out_shape=jax.ShapeDtypeStruct((N, 240), jnp.float32),
    )(*args)
    return out[None]
